# baseline (speedup 1.0000x reference)
import jax
import jax.numpy as jnp
from jax import lax
from jax.experimental import pallas as pl
from jax.experimental.pallas import tpu as pltpu
from jax.experimental.pallas import tpu_sc as plsc

B, C, H, W = 8, 192, 224, 224
C_BLK = 24
C_PAD = 256


def _sc_gather(idx_hbm, table_hbm, out_hbm, idx_v, rows_v, sem):
    wid = lax.axis_index("s") * 2 + lax.axis_index("c")

    @pl.when(wid == 0)
    def _():
        pltpu.sync_copy(idx_hbm, idx_v)
        pltpu.async_copy(table_hbm.at[idx_v], rows_v, sem).wait()
        pltpu.sync_copy(rows_v, out_hbm)


@jax.jit
def _gather_rows(cmd_idx, cmd_embedding):
    mesh = plsc.VectorSubcoreMesh(core_axis_name="c", subcore_axis_name="s")
    return pl.kernel(
        _sc_gather,
        out_type=jax.ShapeDtypeStruct((B, C_PAD), jnp.float32),
        mesh=mesh,
        scratch_types=[
            pltpu.VMEM((B,), jnp.int32),
            pltpu.VMEM((B, C_PAD), jnp.float32),
            pltpu.SemaphoreType.DMA,
        ],
    )(cmd_idx, cmd_embedding)


def _add_kernel(emb_smem, x_ref, o_ref):
    b = pl.program_id(0)
    j = pl.program_id(1)
    for ci in range(C_BLK):
        o_ref[0, ci] = x_ref[0, ci] + emb_smem[b, j * C_BLK + ci]


@jax.jit
def _broadcast_add(x, emb):
    grid = (B, C // C_BLK)
    return pl.pallas_call(
        _add_kernel,
        grid=grid,
        in_specs=[
            pl.BlockSpec(memory_space=pltpu.SMEM),
            pl.BlockSpec((1, C_BLK, H, W), lambda b, j: (b, j, 0, 0)),
        ],
        out_specs=pl.BlockSpec((1, C_BLK, H, W), lambda b, j: (b, j, 0, 0)),
        out_shape=jax.ShapeDtypeStruct((B, C, H, W), jnp.float32),
    )(emb, x)


def kernel(x, cmd_idx, cmd_embedding):
    table = jnp.pad(cmd_embedding, ((0, 0), (0, C_PAD - C)))
    emb = _gather_rows(cmd_idx.astype(jnp.int32), table)
    return _broadcast_add(x, emb)


# speedup vs baseline: 1.0063x; 1.0063x over previous
import jax
import jax.numpy as jnp
from jax import lax
from jax.experimental import pallas as pl
from jax.experimental.pallas import tpu as pltpu
from jax.experimental.pallas import tpu_sc as plsc

B, C, H, W = 8, 192, 224, 224
C_BLK = 48
C_PAD = 256


def _sc_gather(idx_hbm, table_hbm, out_hbm, idx_v, rows_v, sem):
    wid = lax.axis_index("s") * 2 + lax.axis_index("c")

    @pl.when(wid == 0)
    def _():
        pltpu.sync_copy(idx_hbm, idx_v)
        pltpu.async_copy(table_hbm.at[idx_v], rows_v, sem).wait()
        pltpu.sync_copy(rows_v, out_hbm)


@jax.jit
def _gather_rows(cmd_idx, cmd_embedding):
    mesh = plsc.VectorSubcoreMesh(core_axis_name="c", subcore_axis_name="s")
    return pl.kernel(
        _sc_gather,
        out_type=jax.ShapeDtypeStruct((B, C_PAD), jnp.float32),
        mesh=mesh,
        scratch_types=[
            pltpu.VMEM((B,), jnp.int32),
            pltpu.VMEM((B, C_PAD), jnp.float32),
            pltpu.SemaphoreType.DMA,
        ],
    )(cmd_idx, cmd_embedding)


def _add_kernel(emb_smem, x_ref, o_ref):
    b = pl.program_id(0)
    j = pl.program_id(1)
    for ci in range(C_BLK):
        o_ref[0, ci] = x_ref[0, ci] + emb_smem[b, j * C_BLK + ci]


@jax.jit
def _broadcast_add(x, emb):
    grid = (B, C // C_BLK)
    return pl.pallas_call(
        _add_kernel,
        grid=grid,
        in_specs=[
            pl.BlockSpec(memory_space=pltpu.SMEM),
            pl.BlockSpec((1, C_BLK, H, W), lambda b, j: (b, j, 0, 0)),
        ],
        out_specs=pl.BlockSpec((1, C_BLK, H, W), lambda b, j: (b, j, 0, 0)),
        out_shape=jax.ShapeDtypeStruct((B, C, H, W), jnp.float32),
    )(emb, x)


def kernel(x, cmd_idx, cmd_embedding):
    table = jnp.pad(cmd_embedding, ((0, 0), (0, C_PAD - C)))
    emb = _gather_rows(cmd_idx.astype(jnp.int32), table)
    return _broadcast_add(x, emb)


# speedup vs baseline: 1.0100x; 1.0037x over previous
import jax
import jax.numpy as jnp
from jax import lax
from jax.experimental import pallas as pl
from jax.experimental.pallas import tpu as pltpu
from jax.experimental.pallas import tpu_sc as plsc

B, C, H, W = 8, 192, 224, 224
C_BLK = 64
C_PAD = 256


def _sc_gather(idx_hbm, table_hbm, out_hbm, idx_v, rows_v, sem):
    wid = lax.axis_index("s") * 2 + lax.axis_index("c")

    @pl.when(wid == 0)
    def _():
        pltpu.sync_copy(idx_hbm, idx_v)
        pltpu.async_copy(table_hbm.at[idx_v], rows_v, sem).wait()
        pltpu.sync_copy(rows_v, out_hbm)


@jax.jit
def _gather_rows(cmd_idx, cmd_embedding):
    mesh = plsc.VectorSubcoreMesh(core_axis_name="c", subcore_axis_name="s")
    return pl.kernel(
        _sc_gather,
        out_type=jax.ShapeDtypeStruct((B, C_PAD), jnp.float32),
        mesh=mesh,
        scratch_types=[
            pltpu.VMEM((B,), jnp.int32),
            pltpu.VMEM((B, C_PAD), jnp.float32),
            pltpu.SemaphoreType.DMA,
        ],
    )(cmd_idx, cmd_embedding)


def _add_kernel(emb_smem, x_ref, o_ref):
    b = pl.program_id(0)
    j = pl.program_id(1)
    for ci in range(C_BLK):
        o_ref[0, ci] = x_ref[0, ci] + emb_smem[b, j * C_BLK + ci]


@jax.jit
def _broadcast_add(x, emb):
    grid = (B, C // C_BLK)
    return pl.pallas_call(
        _add_kernel,
        grid=grid,
        in_specs=[
            pl.BlockSpec(memory_space=pltpu.SMEM),
            pl.BlockSpec((1, C_BLK, H, W), lambda b, j: (b, j, 0, 0)),
        ],
        out_specs=pl.BlockSpec((1, C_BLK, H, W), lambda b, j: (b, j, 0, 0)),
        out_shape=jax.ShapeDtypeStruct((B, C, H, W), jnp.float32),
    )(emb, x)


def kernel(x, cmd_idx, cmd_embedding):
    table = jnp.pad(cmd_embedding, ((0, 0), (0, C_PAD - C)))
    emb = _gather_rows(cmd_idx.astype(jnp.int32), table)
    return _broadcast_add(x, emb)
